# Initial kernel scaffold; baseline (speedup 1.0000x reference)
import jax
import jax.numpy as jnp
from jax.experimental import pallas as pl


def kernel(inputs, xyz, W, gamma, beta):
    raise NotImplementedError("write your pallas kernel here")


# speedup vs baseline: 12.9822x; 12.9822x over previous
import functools

import jax
import jax.numpy as jnp
from jax import lax
from jax.experimental import pallas as pl
from jax.experimental.pallas import tpu as pltpu
from jax.experimental.pallas import tpu_sc as plsc

B, N, C, K, OUT = 8, 2048, 64, 8, 64
TILE = 256
NT = N // TILE
M_TOTAL = float(B * N * K)
EPS = 1e-3

NC, NS = 2, 16
NW = NC * NS
R_TOTAL = B * N * K
ROWS_PER_W = R_TOTAL // NW
CH = 128
NCH = ROWS_PER_W // CH


def _knn_kernel(xyz_ref, idx_ref):
    b = pl.program_id(0)
    t = pl.program_id(1)
    X = xyz_ref[0]
    xx = jnp.sum(X * X, axis=0, keepdims=True)
    off = pl.multiple_of(t * TILE, TILE)
    xt = xyz_ref[0, :, pl.ds(off, TILE)]
    dotp = lax.dot_general(xt, X, (((0,), (0,)), ((), ())),
                           preferred_element_type=jnp.float32)
    colxx = jnp.sum(xt * xt, axis=0)[:, None]
    pair = dotp + dotp - colxx - xx
    iota = lax.broadcasted_iota(jnp.int32, (TILE, N), 1)
    base = b * N
    for k in range(K):
        m = jnp.max(pair, axis=1, keepdims=True)
        sel = jnp.where(pair == m, iota, N)
        a = jnp.min(sel, axis=1)
        idx_ref[0, k, :] = a + base
        pair = jnp.where(iota == a[:, None], -jnp.inf, pair)


def _sc_gather(table_hbm, idx_hbm, out_hbm, idx_v, buf, sem):
    wid = lax.axis_index("s") * NC + lax.axis_index("c")
    base = wid * ROWS_PER_W
    pltpu.sync_copy(idx_hbm.at[wid], idx_v)

    def body(c, carry):
        cp = pltpu.async_copy(table_hbm.at[idx_v.at[c]], buf, sem)
        cp.wait()
        pltpu.sync_copy(buf, out_hbm.at[pl.ds(base + c * CH, CH)])
        return carry

    lax.fori_loop(0, NCH, body, 0)


def _stats_kernel(g_ref, x_ref, w1_ref, w2_ref, gamma_ref, beta_ref,
                  wq1_ref, wv_ref, bq_ref,
                  p_s, r_s, rt_s, q_s, sg_s, sx_s):
    b = pl.program_id(0)
    t = pl.program_id(1)
    first = jnp.logical_and(b == 0, t == 0)
    last = jnp.logical_and(b == B - 1, t == NT - 1)

    g3 = g_ref[0]
    X = x_ref[0]
    g2 = g3.reshape(K * TILE, C)
    sk = jnp.sum(g3, axis=0)
    cdim = (((0,), (0,)), ((), ()))
    dP = lax.dot_general(g2, g2, cdim, preferred_element_type=jnp.float32)
    dR = lax.dot_general(sk, X, cdim, preferred_element_type=jnp.float32)
    dRt = lax.dot_general(X, sk, cdim, preferred_element_type=jnp.float32)
    dQ = lax.dot_general(X, X, cdim, preferred_element_type=jnp.float32)
    dSg = jnp.sum(g2, axis=0, keepdims=True)
    dSx = jnp.sum(X, axis=0, keepdims=True)

    @pl.when(first)
    def _():
        p_s[:] = dP
        r_s[:] = dR
        rt_s[:] = dRt
        q_s[:] = dQ
        sg_s[:] = dSg
        sx_s[:] = dSx

    @pl.when(jnp.logical_not(first))
    def _():
        p_s[:] += dP
        r_s[:] += dR
        rt_s[:] += dRt
        q_s[:] += dQ
        sg_s[:] += dSg
        sx_s[:] += dSx

    @pl.when(last)
    def _():
        kf = float(K)
        P, R, Rt, Q = p_s[:], r_s[:], rt_s[:], q_s[:]
        Sg, Sx = sg_s[:], sx_s[:]
        G11 = P - R - Rt + kf * Q
        G12 = R - kf * Q
        G21 = Rt - kf * Q
        G22 = kf * Q
        W1, W2 = w1_ref[:], w2_ref[:]
        cd = (((1,), (0,)), ((), ()))
        T1 = (lax.dot_general(W1, G11, cd, preferred_element_type=jnp.float32)
              + lax.dot_general(W2, G21, cd, preferred_element_type=jnp.float32))
        T2 = (lax.dot_general(W1, G12, cd, preferred_element_type=jnp.float32)
              + lax.dot_general(W2, G22, cd, preferred_element_type=jnp.float32))
        esq = (jnp.sum(T1 * W1, axis=1, keepdims=True)
               + jnp.sum(T2 * W2, axis=1, keepdims=True)) / M_TOTAL
        Sd = Sg - kf * Sx
        Sx2 = kf * Sx
        cd2 = (((1,), (1,)), ((), ()))
        mean = (lax.dot_general(W1, Sd, cd2, preferred_element_type=jnp.float32)
                + lax.dot_general(W2, Sx2, cd2,
                                  preferred_element_type=jnp.float32)) / M_TOTAL
        var = esq - mean * mean
        scale = gamma_ref[:] * lax.rsqrt(var + EPS)
        wq1_ref[:] = W1 * scale
        wv_ref[:] = (W2 - W1) * scale
        bq_ref[:] = beta_ref[:] - mean * scale


def _edge_kernel(g_ref, x_ref, wq1_ref, wv_ref, bq_ref, out_ref):
    g2 = g_ref[0].reshape(K * TILE, C)
    cd = (((1,), (1,)), ((), ()))
    A = lax.dot_general(g2, wq1_ref[:], cd,
                        preferred_element_type=jnp.float32)
    A3 = A.reshape(K, TILE, OUT)
    Cx = lax.dot_general(x_ref[0], wv_ref[:], cd,
                         preferred_element_type=jnp.float32) + bq_ref[:]
    v = A3 + Cx[None, :, :]
    v = jnp.where(v >= 0.0, v, 0.2 * v)
    out_ref[0] = jnp.max(v, axis=0)


def kernel(inputs, xyz, W, gamma, beta):
    xyz_pad = jnp.pad(xyz.astype(jnp.float32), ((0, 0), (0, 5), (0, 0)))

    idx = pl.pallas_call(
        _knn_kernel,
        grid=(B, NT),
        in_specs=[pl.BlockSpec((1, 8, N), lambda b, t: (b, 0, 0))],
        out_specs=pl.BlockSpec((1, K, TILE), lambda b, t: (b, 0, t)),
        out_shape=jax.ShapeDtypeStruct((B, K, N), jnp.int32),
    )(xyz_pad)

    idx_w = idx.reshape(NW, NCH, CH)
    table = inputs.reshape(B * N, C)

    mesh = plsc.VectorSubcoreMesh(core_axis_name="c", subcore_axis_name="s")
    gathered = pl.kernel(
        _sc_gather,
        out_type=jax.ShapeDtypeStruct((R_TOTAL, C), jnp.float32),
        mesh=mesh,
        scratch_types=[
            pltpu.VMEM((NCH, CH), jnp.int32),
            pltpu.VMEM((CH, C), jnp.float32),
            pltpu.SemaphoreType.DMA,
        ],
        compiler_params=pltpu.CompilerParams(use_tc_tiling_on_sc=False),
    )(table, idx_w)

    g4 = gathered.reshape(B, K, N, C)
    W1 = W[:, :C]
    W2 = W[:, C:]
    gamma2 = gamma.reshape(OUT, 1)
    beta2 = beta.reshape(OUT, 1)

    wspec = pl.BlockSpec((OUT, C), lambda b, t: (0, 0))
    vspec = pl.BlockSpec((OUT, 1), lambda b, t: (0, 0))
    gspec = pl.BlockSpec((1, K, TILE, C), lambda b, t: (b, 0, t, 0))
    xspec = pl.BlockSpec((1, TILE, C), lambda b, t: (b, t, 0))

    wq1, wv, bq = pl.pallas_call(
        _stats_kernel,
        grid=(B, NT),
        in_specs=[gspec, xspec, wspec, wspec, vspec, vspec],
        out_specs=[
            pl.BlockSpec((OUT, C), lambda b, t: (0, 0)),
            pl.BlockSpec((OUT, C), lambda b, t: (0, 0)),
            pl.BlockSpec((OUT, 1), lambda b, t: (0, 0)),
        ],
        out_shape=[
            jax.ShapeDtypeStruct((OUT, C), jnp.float32),
            jax.ShapeDtypeStruct((OUT, C), jnp.float32),
            jax.ShapeDtypeStruct((OUT, 1), jnp.float32),
        ],
        scratch_shapes=[
            pltpu.VMEM((C, C), jnp.float32),
            pltpu.VMEM((C, C), jnp.float32),
            pltpu.VMEM((C, C), jnp.float32),
            pltpu.VMEM((C, C), jnp.float32),
            pltpu.VMEM((1, C), jnp.float32),
            pltpu.VMEM((1, C), jnp.float32),
        ],
    )(g4, inputs, W1, W2, gamma2, beta2)

    bq_row = bq.reshape(1, OUT)

    out = pl.pallas_call(
        _edge_kernel,
        grid=(B, NT),
        in_specs=[gspec, xspec, wspec, wspec,
                  pl.BlockSpec((1, OUT), lambda b, t: (0, 0))],
        out_specs=pl.BlockSpec((1, TILE, OUT), lambda b, t: (b, t, 0)),
        out_shape=jax.ShapeDtypeStruct((B, N, OUT), jnp.float32),
    )(g4, inputs, wq1, wv, bq_row)

    return out


# speedup vs baseline: 14.0995x; 1.0861x over previous
import functools

import jax
import jax.numpy as jnp
from jax import lax
from jax.experimental import pallas as pl
from jax.experimental.pallas import tpu as pltpu
from jax.experimental.pallas import tpu_sc as plsc

B, N, C, K, OUT = 8, 2048, 64, 8, 64
TILE = 256
NT = N // TILE
TILE1 = 512
NT1 = N // TILE1
M_TOTAL = float(B * N * K)
EPS = 1e-3

NC, NS = 2, 16
NW = NC * NS
R_TOTAL = B * N * K
ROWS_PER_W = R_TOTAL // NW
CH = 128
NCH = ROWS_PER_W // CH


def _knn_kernel(xyz_ref, idx_ref):
    b = pl.program_id(0)
    t = pl.program_id(1)
    X = xyz_ref[0]
    xx = jnp.sum(X * X, axis=0, keepdims=True)
    off = pl.multiple_of(t * TILE1, TILE1)
    xt = xyz_ref[0, :, pl.ds(off, TILE1)]
    dotp = lax.dot_general(xt, X, (((0,), (0,)), ((), ())),
                           preferred_element_type=jnp.float32)
    colxx = jnp.sum(xt * xt, axis=0)[:, None]
    pair = dotp + dotp - colxx - xx
    iota = lax.broadcasted_iota(jnp.int32, (TILE1, N), 1)
    base = b * N
    rows = lax.broadcasted_iota(jnp.int32, (TILE1, N), 0) + off
    idx_ref[0, 0, :] = lax.broadcasted_iota(jnp.int32, (TILE1,), 0) + (off + base)
    pair = jnp.where(iota == rows, -jnp.inf, pair)
    for k in range(1, K):
        m = jnp.max(pair, axis=1, keepdims=True)
        sel = jnp.where(pair == m, iota, N)
        a = jnp.min(sel, axis=1)
        idx_ref[0, k, :] = a + base
        pair = jnp.where(iota == a[:, None], -jnp.inf, pair)


def _sc_gather(table_hbm, idx_hbm, out_hbm, idx_v, buf, sem):
    wid = lax.axis_index("s") * NC + lax.axis_index("c")
    base = wid * ROWS_PER_W
    pltpu.sync_copy(idx_hbm.at[wid], idx_v)

    def body(c, carry):
        cp = pltpu.async_copy(table_hbm.at[idx_v.at[c]], buf, sem)
        cp.wait()
        pltpu.sync_copy(buf, out_hbm.at[pl.ds(base + c * CH, CH)])
        return carry

    lax.fori_loop(0, NCH, body, 0)


def _stats_kernel(g_ref, x_ref, w1_ref, w2_ref, gamma_ref, beta_ref,
                  wq1_ref, wv_ref, bq_ref,
                  p_s, r_s, rt_s, q_s, sg_s, sx_s):
    b = pl.program_id(0)
    t = pl.program_id(1)
    first = jnp.logical_and(b == 0, t == 0)
    last = jnp.logical_and(b == B - 1, t == NT - 1)

    g3 = g_ref[0]
    X = x_ref[0]
    g2 = g3.reshape(K * TILE, C)
    sk = jnp.sum(g3, axis=0)
    cdim = (((0,), (0,)), ((), ()))
    dP = lax.dot_general(g2, g2, cdim, preferred_element_type=jnp.float32)
    dR = lax.dot_general(sk, X, cdim, preferred_element_type=jnp.float32)
    dRt = lax.dot_general(X, sk, cdim, preferred_element_type=jnp.float32)
    dQ = lax.dot_general(X, X, cdim, preferred_element_type=jnp.float32)
    dSg = jnp.sum(g2, axis=0, keepdims=True)
    dSx = jnp.sum(X, axis=0, keepdims=True)

    @pl.when(first)
    def _():
        p_s[:] = dP
        r_s[:] = dR
        rt_s[:] = dRt
        q_s[:] = dQ
        sg_s[:] = dSg
        sx_s[:] = dSx

    @pl.when(jnp.logical_not(first))
    def _():
        p_s[:] += dP
        r_s[:] += dR
        rt_s[:] += dRt
        q_s[:] += dQ
        sg_s[:] += dSg
        sx_s[:] += dSx

    @pl.when(last)
    def _():
        kf = float(K)
        P, R, Rt, Q = p_s[:], r_s[:], rt_s[:], q_s[:]
        Sg, Sx = sg_s[:], sx_s[:]
        G11 = P - R - Rt + kf * Q
        G12 = R - kf * Q
        G21 = Rt - kf * Q
        G22 = kf * Q
        W1, W2 = w1_ref[:], w2_ref[:]
        cd = (((1,), (0,)), ((), ()))
        T1 = (lax.dot_general(W1, G11, cd, preferred_element_type=jnp.float32)
              + lax.dot_general(W2, G21, cd, preferred_element_type=jnp.float32))
        T2 = (lax.dot_general(W1, G12, cd, preferred_element_type=jnp.float32)
              + lax.dot_general(W2, G22, cd, preferred_element_type=jnp.float32))
        esq = (jnp.sum(T1 * W1, axis=1, keepdims=True)
               + jnp.sum(T2 * W2, axis=1, keepdims=True)) / M_TOTAL
        Sd = Sg - kf * Sx
        Sx2 = kf * Sx
        cd2 = (((1,), (1,)), ((), ()))
        mean = (lax.dot_general(W1, Sd, cd2, preferred_element_type=jnp.float32)
                + lax.dot_general(W2, Sx2, cd2,
                                  preferred_element_type=jnp.float32)) / M_TOTAL
        var = esq - mean * mean
        scale = gamma_ref[:] * lax.rsqrt(var + EPS)
        wq1_ref[:] = W1 * scale
        wv_ref[:] = (W2 - W1) * scale
        bq_ref[:] = beta_ref[:] - mean * scale


def _edge_kernel(g_ref, x_ref, wq1_ref, wv_ref, bq_ref, out_ref):
    g2 = g_ref[0].reshape(K * TILE, C)
    cd = (((1,), (1,)), ((), ()))
    A = lax.dot_general(g2, wq1_ref[:], cd,
                        preferred_element_type=jnp.float32)
    A3 = A.reshape(K, TILE, OUT)
    Cx = lax.dot_general(x_ref[0], wv_ref[:], cd,
                         preferred_element_type=jnp.float32) + bq_ref[:]
    v = A3 + Cx[None, :, :]
    v = jnp.where(v >= 0.0, v, 0.2 * v)
    out_ref[0] = jnp.max(v, axis=0)


def kernel(inputs, xyz, W, gamma, beta):
    xyz_pad = jnp.pad(xyz.astype(jnp.float32), ((0, 0), (0, 5), (0, 0)))

    idx = pl.pallas_call(
        _knn_kernel,
        grid=(B, NT1),
        in_specs=[pl.BlockSpec((1, 8, N), lambda b, t: (b, 0, 0))],
        out_specs=pl.BlockSpec((1, K, TILE1), lambda b, t: (b, 0, t)),
        out_shape=jax.ShapeDtypeStruct((B, K, N), jnp.int32),
    )(xyz_pad)

    idx_w = idx.reshape(NW, NCH, CH)
    table = inputs.reshape(B * N, C)

    mesh = plsc.VectorSubcoreMesh(core_axis_name="c", subcore_axis_name="s")
    gathered = pl.kernel(
        _sc_gather,
        out_type=jax.ShapeDtypeStruct((R_TOTAL, C), jnp.float32),
        mesh=mesh,
        scratch_types=[
            pltpu.VMEM((NCH, CH), jnp.int32),
            pltpu.VMEM((CH, C), jnp.float32),
            pltpu.SemaphoreType.DMA,
        ],
        compiler_params=pltpu.CompilerParams(use_tc_tiling_on_sc=False),
    )(table, idx_w)

    g4 = gathered.reshape(B, K, N, C)
    W1 = W[:, :C]
    W2 = W[:, C:]
    gamma2 = gamma.reshape(OUT, 1)
    beta2 = beta.reshape(OUT, 1)

    wspec = pl.BlockSpec((OUT, C), lambda b, t: (0, 0))
    vspec = pl.BlockSpec((OUT, 1), lambda b, t: (0, 0))
    gspec = pl.BlockSpec((1, K, TILE, C), lambda b, t: (b, 0, t, 0))
    xspec = pl.BlockSpec((1, TILE, C), lambda b, t: (b, t, 0))

    wq1, wv, bq = pl.pallas_call(
        _stats_kernel,
        grid=(B, NT),
        in_specs=[gspec, xspec, wspec, wspec, vspec, vspec],
        out_specs=[
            pl.BlockSpec((OUT, C), lambda b, t: (0, 0)),
            pl.BlockSpec((OUT, C), lambda b, t: (0, 0)),
            pl.BlockSpec((OUT, 1), lambda b, t: (0, 0)),
        ],
        out_shape=[
            jax.ShapeDtypeStruct((OUT, C), jnp.float32),
            jax.ShapeDtypeStruct((OUT, C), jnp.float32),
            jax.ShapeDtypeStruct((OUT, 1), jnp.float32),
        ],
        scratch_shapes=[
            pltpu.VMEM((C, C), jnp.float32),
            pltpu.VMEM((C, C), jnp.float32),
            pltpu.VMEM((C, C), jnp.float32),
            pltpu.VMEM((C, C), jnp.float32),
            pltpu.VMEM((1, C), jnp.float32),
            pltpu.VMEM((1, C), jnp.float32),
        ],
    )(g4, inputs, W1, W2, gamma2, beta2)

    bq_row = bq.reshape(1, OUT)

    out = pl.pallas_call(
        _edge_kernel,
        grid=(B, NT),
        in_specs=[gspec, xspec, wspec, wspec,
                  pl.BlockSpec((1, OUT), lambda b, t: (0, 0))],
        out_specs=pl.BlockSpec((1, TILE, OUT), lambda b, t: (b, t, 0)),
        out_shape=jax.ShapeDtypeStruct((B, N, OUT), jnp.float32),
    )(g4, inputs, wq1, wv, bq_row)

    return out


# speedup vs baseline: 15.7989x; 1.1205x over previous
import functools

import jax
import jax.numpy as jnp
from jax import lax
from jax.experimental import pallas as pl
from jax.experimental.pallas import tpu as pltpu
from jax.experimental.pallas import tpu_sc as plsc

B, N, C, K, OUT = 8, 2048, 64, 8, 64
TILE = 256
NT = N // TILE
TILE1 = 512
NT1 = N // TILE1
M_TOTAL = float(B * N * K)
EPS = 1e-3

NC, NS = 2, 16
NW = NC * NS
R_TOTAL = B * N * K
ROWS_PER_W = R_TOTAL // NW
CH = 128
NCH = ROWS_PER_W // CH


def _knn_kernel(xyz_ref, idx_ref):
    b = pl.program_id(0)
    t = pl.program_id(1)
    X = xyz_ref[0]
    xx = jnp.sum(X * X, axis=0, keepdims=True)
    off = pl.multiple_of(t * TILE1, TILE1)
    xt = xyz_ref[0, :, pl.ds(off, TILE1)]
    dotp = lax.dot_general(xt, X, (((0,), (0,)), ((), ())),
                           preferred_element_type=jnp.float32)
    colxx = jnp.sum(xt * xt, axis=0)[:, None]
    iota = lax.broadcasted_iota(jnp.int32, (TILE1, N), 1)
    base = b * N
    rows = lax.broadcasted_iota(jnp.int32, (TILE1, N), 0) + off
    pair = jnp.where(iota == rows, -jnp.inf,
                     dotp + dotp - colxx - xx)
    idx_ref[0, 0, :] = lax.broadcasted_iota(jnp.int32, (TILE1,), 0) + (off + base)
    for k in range(1, K):
        a = jnp.argmax(pair, axis=1).astype(jnp.int32)
        idx_ref[0, k, :] = a + base
        pair = jnp.where(iota == a[:, None], -jnp.inf, pair)


def _sc_gather(table_hbm, idx_hbm, out_hbm, idx_v, buf, sem):
    wid = lax.axis_index("s") * NC + lax.axis_index("c")
    base = wid * ROWS_PER_W
    pltpu.sync_copy(idx_hbm.at[wid], idx_v)

    def body(c, carry):
        cp = pltpu.async_copy(table_hbm.at[idx_v.at[c]], buf, sem)
        cp.wait()
        pltpu.sync_copy(buf, out_hbm.at[pl.ds(base + c * CH, CH)])
        return carry

    lax.fori_loop(0, NCH, body, 0)


def _stats_kernel(g_ref, x_ref, w1_ref, w2_ref, gamma_ref, beta_ref,
                  wq1_ref, wv_ref, bq_ref,
                  p_s, r_s, rt_s, q_s, sg_s, sx_s):
    b = pl.program_id(0)
    t = pl.program_id(1)
    first = jnp.logical_and(b == 0, t == 0)
    last = jnp.logical_and(b == B - 1, t == NT - 1)

    g3 = g_ref[0]
    X = x_ref[0]
    g2 = g3.reshape(K * TILE, C)
    sk = jnp.sum(g3, axis=0)
    cdim = (((0,), (0,)), ((), ()))
    dP = lax.dot_general(g2, g2, cdim, preferred_element_type=jnp.float32)
    dR = lax.dot_general(sk, X, cdim, preferred_element_type=jnp.float32)
    dRt = lax.dot_general(X, sk, cdim, preferred_element_type=jnp.float32)
    dQ = lax.dot_general(X, X, cdim, preferred_element_type=jnp.float32)
    dSg = jnp.sum(g2, axis=0, keepdims=True)
    dSx = jnp.sum(X, axis=0, keepdims=True)

    @pl.when(first)
    def _():
        p_s[:] = dP
        r_s[:] = dR
        rt_s[:] = dRt
        q_s[:] = dQ
        sg_s[:] = dSg
        sx_s[:] = dSx

    @pl.when(jnp.logical_not(first))
    def _():
        p_s[:] += dP
        r_s[:] += dR
        rt_s[:] += dRt
        q_s[:] += dQ
        sg_s[:] += dSg
        sx_s[:] += dSx

    @pl.when(last)
    def _():
        kf = float(K)
        P, R, Rt, Q = p_s[:], r_s[:], rt_s[:], q_s[:]
        Sg, Sx = sg_s[:], sx_s[:]
        G11 = P - R - Rt + kf * Q
        G12 = R - kf * Q
        G21 = Rt - kf * Q
        G22 = kf * Q
        W1, W2 = w1_ref[:], w2_ref[:]
        cd = (((1,), (0,)), ((), ()))
        T1 = (lax.dot_general(W1, G11, cd, preferred_element_type=jnp.float32)
              + lax.dot_general(W2, G21, cd, preferred_element_type=jnp.float32))
        T2 = (lax.dot_general(W1, G12, cd, preferred_element_type=jnp.float32)
              + lax.dot_general(W2, G22, cd, preferred_element_type=jnp.float32))
        esq = (jnp.sum(T1 * W1, axis=1, keepdims=True)
               + jnp.sum(T2 * W2, axis=1, keepdims=True)) / M_TOTAL
        Sd = Sg - kf * Sx
        Sx2 = kf * Sx
        cd2 = (((1,), (1,)), ((), ()))
        mean = (lax.dot_general(W1, Sd, cd2, preferred_element_type=jnp.float32)
                + lax.dot_general(W2, Sx2, cd2,
                                  preferred_element_type=jnp.float32)) / M_TOTAL
        var = esq - mean * mean
        scale = gamma_ref[:] * lax.rsqrt(var + EPS)
        wq1_ref[:] = W1 * scale
        wv_ref[:] = (W2 - W1) * scale
        bq_ref[:] = beta_ref[:] - mean * scale


def _edge_kernel(g_ref, x_ref, wq1_ref, wv_ref, bq_ref, out_ref):
    g2 = g_ref[0].reshape(K * TILE, C)
    cd = (((1,), (1,)), ((), ()))
    A = lax.dot_general(g2, wq1_ref[:], cd,
                        preferred_element_type=jnp.float32)
    A3 = A.reshape(K, TILE, OUT)
    Cx = lax.dot_general(x_ref[0], wv_ref[:], cd,
                         preferred_element_type=jnp.float32) + bq_ref[:]
    v = A3 + Cx[None, :, :]
    v = jnp.where(v >= 0.0, v, 0.2 * v)
    out_ref[0] = jnp.max(v, axis=0)


def kernel(inputs, xyz, W, gamma, beta):
    xyz_pad = jnp.pad(xyz.astype(jnp.float32), ((0, 0), (0, 5), (0, 0)))

    idx = pl.pallas_call(
        _knn_kernel,
        grid=(B, NT1),
        in_specs=[pl.BlockSpec((1, 8, N), lambda b, t: (b, 0, 0))],
        out_specs=pl.BlockSpec((1, K, TILE1), lambda b, t: (b, 0, t)),
        out_shape=jax.ShapeDtypeStruct((B, K, N), jnp.int32),
    )(xyz_pad)

    idx_w = idx.reshape(NW, NCH, CH)
    table = inputs.reshape(B * N, C)

    mesh = plsc.VectorSubcoreMesh(core_axis_name="c", subcore_axis_name="s")
    gathered = pl.kernel(
        _sc_gather,
        out_type=jax.ShapeDtypeStruct((R_TOTAL, C), jnp.float32),
        mesh=mesh,
        scratch_types=[
            pltpu.VMEM((NCH, CH), jnp.int32),
            pltpu.VMEM((CH, C), jnp.float32),
            pltpu.SemaphoreType.DMA,
        ],
        compiler_params=pltpu.CompilerParams(use_tc_tiling_on_sc=False),
    )(table, idx_w)

    g4 = gathered.reshape(B, K, N, C)
    W1 = W[:, :C]
    W2 = W[:, C:]
    gamma2 = gamma.reshape(OUT, 1)
    beta2 = beta.reshape(OUT, 1)

    wspec = pl.BlockSpec((OUT, C), lambda b, t: (0, 0))
    vspec = pl.BlockSpec((OUT, 1), lambda b, t: (0, 0))
    gspec = pl.BlockSpec((1, K, TILE, C), lambda b, t: (b, 0, t, 0))
    xspec = pl.BlockSpec((1, TILE, C), lambda b, t: (b, t, 0))

    wq1, wv, bq = pl.pallas_call(
        _stats_kernel,
        grid=(B, NT),
        in_specs=[gspec, xspec, wspec, wspec, vspec, vspec],
        out_specs=[
            pl.BlockSpec((OUT, C), lambda b, t: (0, 0)),
            pl.BlockSpec((OUT, C), lambda b, t: (0, 0)),
            pl.BlockSpec((OUT, 1), lambda b, t: (0, 0)),
        ],
        out_shape=[
            jax.ShapeDtypeStruct((OUT, C), jnp.float32),
            jax.ShapeDtypeStruct((OUT, C), jnp.float32),
            jax.ShapeDtypeStruct((OUT, 1), jnp.float32),
        ],
        scratch_shapes=[
            pltpu.VMEM((C, C), jnp.float32),
            pltpu.VMEM((C, C), jnp.float32),
            pltpu.VMEM((C, C), jnp.float32),
            pltpu.VMEM((C, C), jnp.float32),
            pltpu.VMEM((1, C), jnp.float32),
            pltpu.VMEM((1, C), jnp.float32),
        ],
    )(g4, inputs, W1, W2, gamma2, beta2)

    bq_row = bq.reshape(1, OUT)

    out = pl.pallas_call(
        _edge_kernel,
        grid=(B, NT),
        in_specs=[gspec, xspec, wspec, wspec,
                  pl.BlockSpec((1, OUT), lambda b, t: (0, 0))],
        out_specs=pl.BlockSpec((1, TILE, OUT), lambda b, t: (b, t, 0)),
        out_shape=jax.ShapeDtypeStruct((B, N, OUT), jnp.float32),
    )(g4, inputs, wq1, wv, bq_row)

    return out


# speedup vs baseline: 18.0790x; 1.1443x over previous
import jax
import jax.numpy as jnp
from jax import lax
from jax.experimental import pallas as pl
from jax.experimental.pallas import tpu as pltpu
from jax.experimental.pallas import tpu_sc as plsc

B, N, C, K, OUT = 8, 2048, 64, 8, 64
HB = B // 2
TILE = 256
NT = N // TILE
TILE1 = 512
NT1 = N // TILE1
M_TOTAL = float(B * N * K)
EPS = 1e-3

NC, NS = 2, 16
NW = NC * NS
RH = HB * N * K
RPW = RH // NW
CH = 128
NCH = RPW // CH


def _make_knn_kernel(off_b):
    def _knn(xyz_ref, idx_ref):
        b = pl.program_id(0)
        t = pl.program_id(1)
        X = xyz_ref[0]
        xx = jnp.sum(X * X, axis=0, keepdims=True)
        off = pl.multiple_of(t * TILE1, TILE1)
        xt = xyz_ref[0, :, pl.ds(off, TILE1)]
        dotp = lax.dot_general(xt, X, (((0,), (0,)), ((), ())),
                               preferred_element_type=jnp.float32)
        colxx = jnp.sum(xt * xt, axis=0)[:, None]
        iota = lax.broadcasted_iota(jnp.int32, (TILE1, N), 1)
        base = (b + off_b) * N
        rows = lax.broadcasted_iota(jnp.int32, (TILE1, N), 0) + off
        pair = jnp.where(iota == rows, -jnp.inf,
                         dotp + dotp - colxx - xx)
        idx_ref[0, 0, :] = (lax.broadcasted_iota(jnp.int32, (TILE1,), 0)
                            + (off + base))
        for k in range(1, K):
            a = jnp.argmax(pair, axis=1).astype(jnp.int32)
            idx_ref[0, k, :] = a + base
            pair = jnp.where(iota == a[:, None], -jnp.inf, pair)
    return _knn


def _sc_gather(table_hbm, idx_hbm, out_hbm, idx_v, buf, sem):
    wid = lax.axis_index("s") * NC + lax.axis_index("c")
    base = wid * RPW
    pltpu.sync_copy(idx_hbm.at[wid], idx_v)

    def body(c, carry):
        cp = pltpu.async_copy(table_hbm.at[idx_v.at[c]], buf, sem)
        cp.wait()
        pltpu.sync_copy(buf, out_hbm.at[pl.ds(base + c * CH, CH)])
        return carry

    lax.fori_loop(0, NCH, body, 0)


def _stats_kernel(g0_ref, g1_ref, x0_ref, x1_ref, w1_ref, w2_ref,
                  gamma_ref, beta_ref, wq1_ref, wv_ref, bq_ref,
                  p_s, r_s, rt_s, q_s, sg_s, sx_s):
    b = pl.program_id(0)
    t = pl.program_id(1)
    first = jnp.logical_and(b == 0, t == 0)
    last = jnp.logical_and(b == HB - 1, t == NT - 1)

    cdim = (((0,), (0,)), ((), ()))

    def moments(g_ref, x_ref):
        g3 = g_ref[0]
        X = x_ref[0]
        g2 = g3.reshape(K * TILE, C)
        sk = jnp.sum(g3, axis=0)
        dP = lax.dot_general(g2, g2, cdim, preferred_element_type=jnp.float32)
        dR = lax.dot_general(sk, X, cdim, preferred_element_type=jnp.float32)
        dRt = lax.dot_general(X, sk, cdim, preferred_element_type=jnp.float32)
        dQ = lax.dot_general(X, X, cdim, preferred_element_type=jnp.float32)
        dSg = jnp.sum(g2, axis=0, keepdims=True)
        dSx = jnp.sum(X, axis=0, keepdims=True)
        return dP, dR, dRt, dQ, dSg, dSx

    m0 = moments(g0_ref, x0_ref)
    m1 = moments(g1_ref, x1_ref)
    dP, dR, dRt, dQ, dSg, dSx = [a + b_ for a, b_ in zip(m0, m1)]

    @pl.when(first)
    def _():
        p_s[:] = dP
        r_s[:] = dR
        rt_s[:] = dRt
        q_s[:] = dQ
        sg_s[:] = dSg
        sx_s[:] = dSx

    @pl.when(jnp.logical_not(first))
    def _():
        p_s[:] += dP
        r_s[:] += dR
        rt_s[:] += dRt
        q_s[:] += dQ
        sg_s[:] += dSg
        sx_s[:] += dSx

    @pl.when(last)
    def _():
        kf = float(K)
        P, R, Rt, Q = p_s[:], r_s[:], rt_s[:], q_s[:]
        Sg, Sx = sg_s[:], sx_s[:]
        G11 = P - R - Rt + kf * Q
        G12 = R - kf * Q
        G21 = Rt - kf * Q
        G22 = kf * Q
        W1, W2 = w1_ref[:], w2_ref[:]
        cd = (((1,), (0,)), ((), ()))
        T1 = (lax.dot_general(W1, G11, cd, preferred_element_type=jnp.float32)
              + lax.dot_general(W2, G21, cd, preferred_element_type=jnp.float32))
        T2 = (lax.dot_general(W1, G12, cd, preferred_element_type=jnp.float32)
              + lax.dot_general(W2, G22, cd, preferred_element_type=jnp.float32))
        esq = (jnp.sum(T1 * W1, axis=1, keepdims=True)
               + jnp.sum(T2 * W2, axis=1, keepdims=True)) / M_TOTAL
        Sd = Sg - kf * Sx
        Sx2 = kf * Sx
        cd2 = (((1,), (1,)), ((), ()))
        mean = (lax.dot_general(W1, Sd, cd2, preferred_element_type=jnp.float32)
                + lax.dot_general(W2, Sx2, cd2,
                                  preferred_element_type=jnp.float32)) / M_TOTAL
        var = esq - mean * mean
        scale = gamma_ref[:] * lax.rsqrt(var + EPS)
        wq1_ref[:] = W1 * scale
        wv_ref[:] = (W2 - W1) * scale
        bq_ref[:] = beta_ref[:] - mean * scale


def _edge_kernel(g0_ref, g1_ref, x0_ref, x1_ref, wq1_ref, wv_ref, bq_ref,
                 o0_ref, o1_ref):
    cd = (((1,), (1,)), ((), ()))

    def half(g_ref, x_ref, o_ref):
        g2 = g_ref[0].reshape(K * TILE, C)
        A = lax.dot_general(g2, wq1_ref[:], cd,
                            preferred_element_type=jnp.float32)
        A3 = A.reshape(K, TILE, OUT)
        Cx = lax.dot_general(x_ref[0], wv_ref[:], cd,
                             preferred_element_type=jnp.float32) + bq_ref[:]
        v = A3 + Cx[None, :, :]
        v = jnp.where(v >= 0.0, v, 0.2 * v)
        o_ref[0] = jnp.max(v, axis=0)

    half(g0_ref, x0_ref, o0_ref)
    half(g1_ref, x1_ref, o1_ref)


def _half_knn(xyz_pad, off_b):
    return pl.pallas_call(
        _make_knn_kernel(off_b),
        grid=(HB, NT1),
        in_specs=[pl.BlockSpec((1, 8, N), lambda b, t: (b, 0, 0))],
        out_specs=pl.BlockSpec((1, K, TILE1), lambda b, t: (b, 0, t)),
        out_shape=jax.ShapeDtypeStruct((HB, K, N), jnp.int32),
    )(xyz_pad[off_b:off_b + HB])


def _half_gather(table, idx_h):
    mesh = plsc.VectorSubcoreMesh(core_axis_name="c", subcore_axis_name="s")
    return pl.kernel(
        _sc_gather,
        out_type=jax.ShapeDtypeStruct((RH, C), jnp.float32),
        mesh=mesh,
        scratch_types=[
            pltpu.VMEM((NCH, CH), jnp.int32),
            pltpu.VMEM((CH, C), jnp.float32),
            pltpu.SemaphoreType.DMA,
        ],
        compiler_params=pltpu.CompilerParams(use_tc_tiling_on_sc=False),
    )(table, idx_h.reshape(NW, NCH, CH))


def kernel(inputs, xyz, W, gamma, beta):
    xyz_pad = jnp.pad(xyz.astype(jnp.float32), ((0, 0), (0, 5), (0, 0)))
    table = inputs.reshape(B * N, C)

    idx0 = _half_knn(xyz_pad, 0)
    g0 = _half_gather(table, idx0)
    idx1 = _half_knn(xyz_pad, HB)
    g1 = _half_gather(table, idx1)

    g40 = g0.reshape(HB, K, N, C)
    g41 = g1.reshape(HB, K, N, C)
    W1 = W[:, :C]
    W2 = W[:, C:]
    gamma2 = gamma.reshape(OUT, 1)
    beta2 = beta.reshape(OUT, 1)

    wspec = pl.BlockSpec((OUT, C), lambda b, t: (0, 0))
    vspec = pl.BlockSpec((OUT, 1), lambda b, t: (0, 0))
    gspec = pl.BlockSpec((1, K, TILE, C), lambda b, t: (b, 0, t, 0))
    x0spec = pl.BlockSpec((1, TILE, C), lambda b, t: (b, t, 0))
    x1spec = pl.BlockSpec((1, TILE, C), lambda b, t: (b + HB, t, 0))

    wq1, wv, bq = pl.pallas_call(
        _stats_kernel,
        grid=(HB, NT),
        in_specs=[gspec, gspec, x0spec, x1spec, wspec, wspec, vspec, vspec],
        out_specs=[
            pl.BlockSpec((OUT, C), lambda b, t: (0, 0)),
            pl.BlockSpec((OUT, C), lambda b, t: (0, 0)),
            pl.BlockSpec((OUT, 1), lambda b, t: (0, 0)),
        ],
        out_shape=[
            jax.ShapeDtypeStruct((OUT, C), jnp.float32),
            jax.ShapeDtypeStruct((OUT, C), jnp.float32),
            jax.ShapeDtypeStruct((OUT, 1), jnp.float32),
        ],
        scratch_shapes=[
            pltpu.VMEM((C, C), jnp.float32),
            pltpu.VMEM((C, C), jnp.float32),
            pltpu.VMEM((C, C), jnp.float32),
            pltpu.VMEM((C, C), jnp.float32),
            pltpu.VMEM((1, C), jnp.float32),
            pltpu.VMEM((1, C), jnp.float32),
        ],
    )(g40, g41, inputs, inputs, W1, W2, gamma2, beta2)

    bq_row = bq.reshape(1, OUT)

    o0, o1 = pl.pallas_call(
        _edge_kernel,
        grid=(HB, NT),
        in_specs=[gspec, gspec, x0spec, x1spec, wspec, wspec,
                  pl.BlockSpec((1, OUT), lambda b, t: (0, 0))],
        out_specs=[pl.BlockSpec((1, TILE, OUT), lambda b, t: (b, t, 0)),
                   pl.BlockSpec((1, TILE, OUT), lambda b, t: (b, t, 0))],
        out_shape=[jax.ShapeDtypeStruct((HB, N, OUT), jnp.float32),
                   jax.ShapeDtypeStruct((HB, N, OUT), jnp.float32)],
    )(g40, g41, inputs, inputs, wq1, wv, bq_row)

    return jnp.concatenate([o0, o1], axis=0)
